# baseline (speedup 1.0000x reference)
import functools

import jax
import jax.numpy as jnp
from jax import lax
from jax.experimental import pallas as pl
from jax.experimental.pallas import tpu as pltpu

_SMOOTH = 0.1
_RB = 8
_NBUF = 16


def _loss_body(t_ref, x_hbm, out_ref, buf, sems, *, c, rb, nrows, nblocks):
    g = pl.program_id(0)

    def _issue(blk, slot):
        pltpu.make_async_copy(
            x_hbm.at[pl.ds(blk * rb, rb), :], buf.at[slot], sems.at[slot]
        ).start()

    @pl.when(g == 0)
    def _warmup():
        out_ref[0, 0] = 0.0
        for b in range(min(_NBUF, nblocks)):
            _issue(b, b)

    a = _SMOOTH / (c - 1)
    bw = (1.0 - _SMOOTH) - a
    col = lax.broadcasted_iota(jnp.int32, (rb, c), 1)

    for b in range(min(_NBUF, nblocks)):
        blk = g * min(_NBUF, nblocks) + b
        pltpu.make_async_copy(
            x_hbm.at[pl.ds(blk * rb, rb), :], buf.at[b], sems.at[b]
        ).wait()

        x = buf[b]
        t_col = t_ref[0, b * rb:(b + 1) * rb, :]
        p = jnp.sum(jnp.where(col == t_col, x, 0.0), axis=1, keepdims=True)
        s_tot = jnp.sum(x, axis=1, keepdims=True)
        m = jnp.max(x, axis=1, keepdims=True)
        e = jnp.exp(x - m)
        lse = m + jnp.log(jnp.sum(e, axis=1, keepdims=True))

        row_loss = a * (c * lse - s_tot) + bw * (lse - p)
        out_ref[0, 0] += jnp.sum(row_loss) / nrows

        @pl.when(blk + _NBUF < nblocks)
        def _refill():
            _issue(blk + _NBUF, b)


def kernel(pred, target):
    nrows, c = pred.shape
    rpg = _RB * _NBUF
    rb = _RB if nrows % rpg == 0 else nrows
    nblocks = nrows // rb
    nsteps = nblocks // _NBUF if nrows % rpg == 0 else 1
    if nrows % rpg != 0:
        nblocks, nsteps = 1, 1

    t3 = target.astype(jnp.int32).reshape(nsteps, nrows // nsteps, 1)

    out = pl.pallas_call(
        functools.partial(_loss_body, c=c, rb=rb, nrows=nrows,
                          nblocks=nblocks),
        grid=(nsteps,),
        in_specs=[
            pl.BlockSpec((1, nrows // nsteps, 1), lambda g: (g, 0, 0)),
            pl.BlockSpec(memory_space=pltpu.MemorySpace.HBM),
        ],
        out_specs=pl.BlockSpec(memory_space=pltpu.SMEM),
        out_shape=jax.ShapeDtypeStruct((1, 1), jnp.float32),
        scratch_shapes=[
            pltpu.VMEM((_NBUF, rb, c), jnp.float32),
            pltpu.SemaphoreType.DMA((_NBUF,)),
        ],
        compiler_params=pltpu.CompilerParams(
            dimension_semantics=("arbitrary",),
        ),
    )(t3, pred)
    return out.reshape(())
